# Initial kernel scaffold; baseline (speedup 1.0000x reference)
import jax
import jax.numpy as jnp
from jax.experimental import pallas as pl


def kernel(input_vectors, target_words, internal_embeddings, codes_table, points_table, path_lengths_table):
    raise NotImplementedError("write your pallas kernel here")


# speedup vs baseline: 3.6231x; 3.6231x over previous
import functools

import jax
import jax.numpy as jnp
from jax import lax
from jax.experimental import pallas as pl
from jax.experimental.pallas import tpu as pltpu
from jax.experimental.pallas import tpu_sc as plsc

_V = 100000
_D = 64
_B = 16384
_L = 24
_NI = _V - 1

_NC = 2
_NS = 16
_NW = _NC * _NS
_BW = _B // _NW
_C = 32
_NCHUNK = _BW // _C
_CL = _C * _L

_MASK_DOT = 40.0
_MASK_CODE = 1.0


def _sc_body(iv_hbm, tw_hbm, emb_hbm, codes_hbm, pts_hbm, plen_hbm,
             dots_out, codes_out,
             tw_v, plen_v, pts_v, pts_f, codes_v, iv_v, emb_v,
             dout_v, sem):
    wid = lax.axis_index("s") * _NC + lax.axis_index("c")
    base = wid * _BW
    pltpu.sync_copy(tw_hbm.at[pl.ds(base, _BW)], tw_v)
    pltpu.async_copy(plen_hbm.at[tw_v], plen_v, sem).wait()

    lane = lax.iota(jnp.int32, 16)
    m15 = lane == 15

    def chunk_body(ch, carry):
        coff = ch * _C
        goff = base + coff
        idx = tw_v.at[pl.ds(coff, _C)]
        cp_c = pltpu.async_copy(codes_hbm.at[idx], codes_v, sem)
        cp_p = pltpu.async_copy(pts_hbm.at[idx], pts_v, sem)
        pltpu.sync_copy(iv_hbm.at[pl.ds(goff, _C)], iv_v)
        cp_c.wait()
        cp_p.wait()

        def flat_body(i, carry2):
            fb = i * _L
            pts_f[pl.ds(fb, 16)] = pts_v[i, pl.ds(0, 16)]
            pts_f[pl.ds(fb + 8, 16)] = pts_v[i, pl.ds(8, 16)]
            return carry2

        lax.fori_loop(0, _C, flat_body, 0)
        pltpu.async_copy(emb_hbm.at[pts_f], emb_v, sem).wait()

        def samp_body(i, carry2):
            plen = plsc.load_gather(
                plen_v, [jnp.full((16,), coff + i, jnp.int32)])
            iv0 = iv_v[i, pl.ds(0, 16)]
            iv1 = iv_v[i, pl.ds(16, 16)]
            iv2 = iv_v[i, pl.ds(32, 16)]
            iv3 = iv_v[i, pl.ds(48, 16)]
            dbase = i * _L
            for l in range(_L):
                e0 = emb_v[dbase + l, pl.ds(0, 16)]
                e1 = emb_v[dbase + l, pl.ds(16, 16)]
                e2 = emb_v[dbase + l, pl.ds(32, 16)]
                e3 = emb_v[dbase + l, pl.ds(48, 16)]
                acc = iv0 * e0 + iv1 * e1 + iv2 * e2 + iv3 * e3
                csum = plsc.cumsum(acc)
                plsc.store_scatter(
                    dout_v, [jnp.full((16,), dbase + l, jnp.int32)], csum,
                    mask=m15)
            m0 = lane < plen
            m1 = (lane + 8) < plen
            d0 = dout_v[pl.ds(dbase, 16)]
            d1 = dout_v[pl.ds(dbase + 8, 16)]
            dout_v[pl.ds(dbase, 16)] = jnp.where(m0, d0, _MASK_DOT)
            dout_v[pl.ds(dbase + 8, 16)] = jnp.where(m1, d1, _MASK_DOT)
            c0 = codes_v[i, pl.ds(0, 16)]
            c1 = codes_v[i, pl.ds(8, 16)]
            codes_v[i, pl.ds(0, 16)] = jnp.where(m0, c0, _MASK_CODE)
            codes_v[i, pl.ds(8, 16)] = jnp.where(m1, c1, _MASK_CODE)
            return carry2

        lax.fori_loop(0, _C, samp_body, 0)
        pltpu.sync_copy(dout_v, dots_out.at[pl.ds(goff * _L, _CL)])
        pltpu.sync_copy(codes_v, codes_out.at[pl.ds(goff, _C)])
        return carry

    lax.fori_loop(0, _NCHUNK, chunk_body, 0)


_sc_dots = functools.partial(
    pl.kernel,
    mesh=plsc.VectorSubcoreMesh(core_axis_name="c", subcore_axis_name="s"),
    out_type=(jax.ShapeDtypeStruct((_B * _L,), jnp.float32),
              jax.ShapeDtypeStruct((_B, _L), jnp.float32)),
    scratch_types=[
        pltpu.VMEM((_BW,), jnp.int32),
        pltpu.VMEM((_BW,), jnp.int32),
        pltpu.VMEM((_C, _L), jnp.int32),
        pltpu.VMEM((_CL,), jnp.int32),
        pltpu.VMEM((_C, _L), jnp.float32),
        pltpu.VMEM((_C, _D), jnp.float32),
        pltpu.VMEM((_CL, _D), jnp.float32),
        pltpu.VMEM((_CL,), jnp.float32),
        pltpu.SemaphoreType.DMA,
    ],
    compiler_params=pltpu.CompilerParams(needs_layout_passes=False,
                                         use_tc_tiling_on_sc=False),
)(_sc_body)


def _loss_body(d_ref, c_ref, o_ref):
    d = d_ref[...]
    c = c_ref[...]
    s = 1.0 / (1.0 + jnp.exp(-d))
    loss = c * jnp.log(s + 1e-10) + (1.0 - c) * jnp.log(1.0 - s + 1e-10)
    o_ref[0, 0] = -jnp.sum(loss)


def kernel(input_vectors, target_words, internal_embeddings, codes_table,
           points_table, path_lengths_table):
    tw = target_words.astype(jnp.int32)
    pts = points_table.astype(jnp.int32)
    plens = path_lengths_table.astype(jnp.int32)
    dots, codes = _sc_dots(input_vectors, tw, internal_embeddings,
                           codes_table, pts, plens)
    r = (_B * _L) // 128
    total = pl.pallas_call(
        _loss_body,
        out_shape=jax.ShapeDtypeStruct((1, 1), jnp.float32),
        out_specs=pl.BlockSpec(memory_space=pltpu.SMEM),
    )(dots.reshape(r, 128), codes.reshape(r, 128))
    return total[0, 0] / _B


# speedup vs baseline: 4.0178x; 1.1089x over previous
import functools

import jax
import jax.numpy as jnp
from jax import lax
from jax.experimental import pallas as pl
from jax.experimental.pallas import tpu as pltpu
from jax.experimental.pallas import tpu_sc as plsc

_V = 100000
_D = 64
_B = 16384
_L = 24
_NI = _V - 1

_NC = 2
_NS = 16
_NW = _NC * _NS
_BW = _B // _NW
_C = 32
_NCHUNK = _BW // _C
_CL = _C * _L

_MASK_DOT = 40.0
_MASK_CODE = 1.0


def _sc_body(iv_hbm, tw_hbm, emb_hbm, codes_hbm, pts_hbm, plen_hbm,
             dots_out, codes_out,
             tw_v, plen_v, pts, cod, iv, ptsf, emb, dout, semi, seme):
    wid = lax.axis_index("s") * _NC + lax.axis_index("c")
    base = wid * _BW
    pltpu.sync_copy(tw_hbm.at[pl.ds(base, _BW)], tw_v)
    pltpu.async_copy(plen_hbm.at[tw_v], plen_v, seme[0]).wait()

    lane = lax.iota(jnp.int32, 16)
    m15 = lane == 15


    def p_stage(ch, k4, k2):
        coff = ch * _C
        idx = tw_v.at[pl.ds(coff, _C)]
        pltpu.async_copy(codes_hbm.at[idx], cod[k4], semi[k2])
        pltpu.async_copy(pts_hbm.at[idx], pts[k4], semi[k2])
        pltpu.async_copy(iv_hbm.at[pl.ds(base + coff, _C)], iv[k4], semi[k2])

    def r_stage(ch, k4, k2):
        coff = ch * _C
        idx = tw_v.at[pl.ds(coff, _C)]
        pltpu.make_async_copy(codes_hbm.at[idx], cod[k4], semi[k2]).wait()
        pltpu.make_async_copy(pts_hbm.at[idx], pts[k4], semi[k2]).wait()
        pltpu.make_async_copy(
            iv_hbm.at[pl.ds(base + coff, _C)], iv[k4], semi[k2]).wait()

        def flat_body(i, carry2):
            fb = i * _L
            ptsf[k2][pl.ds(fb, 16)] = pts[k4][i, pl.ds(0, 16)]
            ptsf[k2][pl.ds(fb + 8, 16)] = pts[k4][i, pl.ds(8, 16)]
            return carry2

        lax.fori_loop(0, _C, flat_body, 0)
        pltpu.async_copy(emb_hbm.at[ptsf[k2]], emb[k2], seme[k2])

    def c_stage(ch, k4, k2):
        coff = ch * _C
        goff = base + coff
        pltpu.make_async_copy(emb_hbm.at[ptsf[k2]], emb[k2], seme[k2]).wait()
        emb_v, iv_v, codes_v, dout_v = emb[k2], iv[k4], cod[k4], dout[k2]

        def samp_body(i, carry2):
            plen = plsc.load_gather(
                plen_v, [jnp.full((16,), coff + i, jnp.int32)])
            iv0 = iv_v[i, pl.ds(0, 16)]
            iv1 = iv_v[i, pl.ds(16, 16)]
            iv2 = iv_v[i, pl.ds(32, 16)]
            iv3 = iv_v[i, pl.ds(48, 16)]
            dbase = i * _L
            for l in range(_L):
                e0 = emb_v[dbase + l, pl.ds(0, 16)]
                e1 = emb_v[dbase + l, pl.ds(16, 16)]
                e2 = emb_v[dbase + l, pl.ds(32, 16)]
                e3 = emb_v[dbase + l, pl.ds(48, 16)]
                acc = iv0 * e0 + iv1 * e1 + iv2 * e2 + iv3 * e3
                csum = plsc.cumsum(acc)
                plsc.store_scatter(
                    dout_v, [jnp.full((16,), dbase + l, jnp.int32)], csum,
                    mask=m15)
            m0 = lane < plen
            m1 = (lane + 8) < plen
            d0 = dout_v[pl.ds(dbase, 16)]
            d1 = dout_v[pl.ds(dbase + 8, 16)]
            dout_v[pl.ds(dbase, 16)] = jnp.where(m0, d0, _MASK_DOT)
            dout_v[pl.ds(dbase + 8, 16)] = jnp.where(m1, d1, _MASK_DOT)
            c0 = codes_v[i, pl.ds(0, 16)]
            c1 = codes_v[i, pl.ds(8, 16)]
            codes_v[i, pl.ds(0, 16)] = jnp.where(m0, c0, _MASK_CODE)
            codes_v[i, pl.ds(8, 16)] = jnp.where(m1, c1, _MASK_CODE)
            return carry2

        lax.fori_loop(0, _C, samp_body, 0)
        pltpu.sync_copy(dout_v, dots_out.at[pl.ds(goff * _L, _CL)])
        pltpu.sync_copy(codes_v, codes_out.at[pl.ds(goff, _C)])

    p_stage(0, 0, 0)
    p_stage(1, 1, 1)
    r_stage(0, 0, 0)

    def body(kk, carry):
        ch0 = kk * 4
        for j in range(4):
            ch = ch0 + j
            r_stage(ch + 1, (j + 1) % 4, (j + 1) % 2)
            p_stage(ch + 2, (j + 2) % 4, j % 2)
            c_stage(ch, j, j % 2)
        return carry

    lax.fori_loop(0, (_NCHUNK - 4) // 4, body, 0)
    for ch in range(_NCHUNK - 4, _NCHUNK):
        if ch + 1 < _NCHUNK:
            r_stage(ch + 1, (ch + 1) % 4, (ch + 1) % 2)
        if ch + 2 < _NCHUNK:
            p_stage(ch + 2, (ch + 2) % 4, ch % 2)
        c_stage(ch, ch % 4, ch % 2)


_sc_dots = functools.partial(
    pl.kernel,
    mesh=plsc.VectorSubcoreMesh(core_axis_name="c", subcore_axis_name="s"),
    out_type=(jax.ShapeDtypeStruct((_B * _L,), jnp.float32),
              jax.ShapeDtypeStruct((_B, _L), jnp.float32)),
    scratch_types=[
        pltpu.VMEM((_BW,), jnp.int32),
        pltpu.VMEM((_BW,), jnp.int32),
        [pltpu.VMEM((_C, _L), jnp.int32)] * 4,
        [pltpu.VMEM((_C, _L), jnp.float32)] * 4,
        [pltpu.VMEM((_C, _D), jnp.float32)] * 4,
        [pltpu.VMEM((_CL,), jnp.int32)] * 2,
        [pltpu.VMEM((_CL, _D), jnp.float32)] * 2,
        [pltpu.VMEM((_CL,), jnp.float32)] * 2,
        [pltpu.SemaphoreType.DMA] * 2,
        [pltpu.SemaphoreType.DMA] * 2,
    ],
    compiler_params=pltpu.CompilerParams(needs_layout_passes=False,
                                         use_tc_tiling_on_sc=False),
)(_sc_body)


def _loss_body(d_ref, c_ref, o_ref):
    d = d_ref[...]
    c = c_ref[...]
    s = 1.0 / (1.0 + jnp.exp(-d))
    loss = c * jnp.log(s + 1e-10) + (1.0 - c) * jnp.log(1.0 - s + 1e-10)
    o_ref[0, 0] = -jnp.sum(loss)


def kernel(input_vectors, target_words, internal_embeddings, codes_table,
           points_table, path_lengths_table):
    tw = target_words.astype(jnp.int32)
    pts = points_table.astype(jnp.int32)
    plens = path_lengths_table.astype(jnp.int32)
    dots, codes = _sc_dots(input_vectors, tw, internal_embeddings,
                           codes_table, pts, plens)
    r = (_B * _L) // 128
    total = pl.pallas_call(
        _loss_body,
        out_shape=jax.ShapeDtypeStruct((1, 1), jnp.float32),
        out_specs=pl.BlockSpec(memory_space=pltpu.SMEM),
    )(dots.reshape(r, 128), codes.reshape(r, 128))
    return total[0, 0] / _B


# speedup vs baseline: 6.5826x; 1.6383x over previous
import functools

import jax
import jax.numpy as jnp
from jax import lax
from jax.experimental import pallas as pl
from jax.experimental.pallas import tpu as pltpu
from jax.experimental.pallas import tpu_sc as plsc

_V = 100000
_D = 64
_B = 16384
_L = 24
_NI = _V - 1

_NC = 2
_NS = 16
_NW = _NC * _NS
_BW = _B // _NW
_C = 32
_NCHUNK = _BW // _C
_CL = _C * _L

_MASK_DOT = 40.0
_MASK_CODE = 1.0


def _sc_body(iv_hbm, tw_hbm, emb_hbm, codes_hbm, pts_hbm, plen_hbm,
             dots_out, codes_out,
             tw_v, plen_v, pts, cod, iv, ptsf, emb, dout, semi, seme):
    wid = lax.axis_index("s") * _NC + lax.axis_index("c")
    base = wid * _BW
    pltpu.sync_copy(tw_hbm.at[pl.ds(base, _BW)], tw_v)
    pltpu.async_copy(plen_hbm.at[tw_v], plen_v, seme[0]).wait()

    lane = lax.iota(jnp.int32, 16)
    m15 = lane == 15


    def p_stage(ch, k4, k2):
        coff = ch * _C
        idx = tw_v.at[pl.ds(coff, _C)]
        pltpu.async_copy(codes_hbm.at[idx], cod[k4], semi[k2])
        pltpu.async_copy(pts_hbm.at[idx], pts[k4], semi[k2])
        pltpu.async_copy(iv_hbm.at[pl.ds(base + coff, _C)], iv[k4], semi[k2])

    def r_stage(ch, k4, k2):
        coff = ch * _C
        idx = tw_v.at[pl.ds(coff, _C)]
        pltpu.make_async_copy(codes_hbm.at[idx], cod[k4], semi[k2]).wait()
        pltpu.make_async_copy(pts_hbm.at[idx], pts[k4], semi[k2]).wait()
        pltpu.make_async_copy(
            iv_hbm.at[pl.ds(base + coff, _C)], iv[k4], semi[k2]).wait()

        def flat_body(i, carry2):
            fb = i * _L
            ptsf[k2][pl.ds(fb, 16)] = pts[k4][i, pl.ds(0, 16)]
            ptsf[k2][pl.ds(fb + 8, 16)] = pts[k4][i, pl.ds(8, 16)]
            return carry2

        lax.fori_loop(0, _C, flat_body, 0)
        pltpu.async_copy(emb_hbm.at[ptsf[k2]], emb[k2], seme[k2])

    def c_stage(ch, k4, k2):
        coff = ch * _C
        goff = base + coff
        pltpu.make_async_copy(emb_hbm.at[ptsf[k2]], emb[k2], seme[k2]).wait()
        emb_v, iv_v, codes_v, dout_v = emb[k2], iv[k4], cod[k4], dout[k2]

        splat15 = jnp.full((16,), 15, jnp.int32)

        def samp_body(i, carry2):
            plen = plsc.load_gather(
                plen_v, [jnp.full((16,), coff + i, jnp.int32)])
            iv0 = iv_v[i, pl.ds(0, 16)]
            iv1 = iv_v[i, pl.ds(16, 16)]
            iv2 = iv_v[i, pl.ds(32, 16)]
            iv3 = iv_v[i, pl.ds(48, 16)]
            dbase = i * _L
            res0 = jnp.full((16,), _MASK_DOT)
            res1 = jnp.full((16,), _MASK_DOT)
            for l in range(_L):
                e0 = emb_v[dbase + l, pl.ds(0, 16)]
                e1 = emb_v[dbase + l, pl.ds(16, 16)]
                e2 = emb_v[dbase + l, pl.ds(32, 16)]
                e3 = emb_v[dbase + l, pl.ds(48, 16)]
                acc = (iv0 * e0 + iv1 * e1) + (iv2 * e2 + iv3 * e3)
                csum = plsc.cumsum(acc)
                tot = lax.gather(
                    csum, splat15[:, None],
                    lax.GatherDimensionNumbers(
                        offset_dims=(), collapsed_slice_dims=(0,),
                        start_index_map=(0,)),
                    (1,), mode=lax.GatherScatterMode.PROMISE_IN_BOUNDS)
                if l < 16:
                    res0 = jnp.where(lane == l, tot, res0)
                else:
                    res1 = jnp.where(lane == l - 16, tot, res1)
            m0 = lane < plen
            mh = (lane + 16) < plen
            dout_v[pl.ds(dbase, 16)] = jnp.where(m0, res0, _MASK_DOT)
            dout_v[pl.ds(dbase + 16, 16)] = jnp.where(mh, res1, _MASK_DOT)
            m1 = (lane + 8) < plen
            c0 = codes_v[i, pl.ds(0, 16)]
            c1 = codes_v[i, pl.ds(8, 16)]
            codes_v[i, pl.ds(0, 16)] = jnp.where(m0, c0, _MASK_CODE)
            codes_v[i, pl.ds(8, 16)] = jnp.where(m1, c1, _MASK_CODE)
            return carry2

        lax.fori_loop(0, _C, samp_body, 0)
        pltpu.sync_copy(dout_v.at[pl.ds(0, _CL)],
                        dots_out.at[pl.ds(goff * _L, _CL)])
        pltpu.sync_copy(codes_v, codes_out.at[pl.ds(goff, _C)])

    p_stage(0, 0, 0)
    p_stage(1, 1, 1)
    r_stage(0, 0, 0)

    def body(kk, carry):
        ch0 = kk * 4
        for j in range(4):
            ch = ch0 + j
            r_stage(ch + 1, (j + 1) % 4, (j + 1) % 2)
            p_stage(ch + 2, (j + 2) % 4, j % 2)
            c_stage(ch, j, j % 2)
        return carry

    lax.fori_loop(0, (_NCHUNK - 4) // 4, body, 0)
    for ch in range(_NCHUNK - 4, _NCHUNK):
        if ch + 1 < _NCHUNK:
            r_stage(ch + 1, (ch + 1) % 4, (ch + 1) % 2)
        if ch + 2 < _NCHUNK:
            p_stage(ch + 2, (ch + 2) % 4, ch % 2)
        c_stage(ch, ch % 4, ch % 2)


_sc_dots = functools.partial(
    pl.kernel,
    mesh=plsc.VectorSubcoreMesh(core_axis_name="c", subcore_axis_name="s"),
    out_type=(jax.ShapeDtypeStruct((_B * _L,), jnp.float32),
              jax.ShapeDtypeStruct((_B, _L), jnp.float32)),
    scratch_types=[
        pltpu.VMEM((_BW,), jnp.int32),
        pltpu.VMEM((_BW,), jnp.int32),
        [pltpu.VMEM((_C, _L), jnp.int32)] * 4,
        [pltpu.VMEM((_C, _L), jnp.float32)] * 4,
        [pltpu.VMEM((_C, _D), jnp.float32)] * 4,
        [pltpu.VMEM((_CL,), jnp.int32)] * 2,
        [pltpu.VMEM((_CL, _D), jnp.float32)] * 2,
        [pltpu.VMEM((_CL + 16,), jnp.float32)] * 2,
        [pltpu.SemaphoreType.DMA] * 2,
        [pltpu.SemaphoreType.DMA] * 2,
    ],
    compiler_params=pltpu.CompilerParams(needs_layout_passes=False,
                                         use_tc_tiling_on_sc=False),
)(_sc_body)


def _loss_body(d_ref, c_ref, o_ref):
    d = d_ref[...]
    c = c_ref[...]
    s = 1.0 / (1.0 + jnp.exp(-d))
    loss = c * jnp.log(s + 1e-10) + (1.0 - c) * jnp.log(1.0 - s + 1e-10)
    o_ref[0, 0] = -jnp.sum(loss)


def kernel(input_vectors, target_words, internal_embeddings, codes_table,
           points_table, path_lengths_table):
    tw = target_words.astype(jnp.int32)
    pts = points_table.astype(jnp.int32)
    plens = path_lengths_table.astype(jnp.int32)
    dots, codes = _sc_dots(input_vectors, tw, internal_embeddings,
                           codes_table, pts, plens)
    r = (_B * _L) // 128
    total = pl.pallas_call(
        _loss_body,
        out_shape=jax.ShapeDtypeStruct((1, 1), jnp.float32),
        out_specs=pl.BlockSpec(memory_space=pltpu.SMEM),
    )(dots.reshape(r, 128), codes.reshape(r, 128))
    return total[0, 0] / _B


# speedup vs baseline: 8.2442x; 1.2524x over previous
import functools

import jax
import jax.numpy as jnp
from jax import lax
from jax.experimental import pallas as pl
from jax.experimental.pallas import tpu as pltpu
from jax.experimental.pallas import tpu_sc as plsc

_V = 100000
_D = 64
_B = 16384
_L = 24
_NI = _V - 1

_NC = 2
_NS = 16
_NW = _NC * _NS
_BW = _B // _NW
_C = 32
_NCHUNK = _BW // _C
_CL = _C * _L

_MASK_DOT = 40.0
_MASK_CODE = 1.0


def _sc_body(iv_hbm, tw_hbm, emb_hbm, codes_hbm, pts_hbm, plen_hbm,
             dots_out, codes_out,
             tw_v, plen_v, idx_v, pts, cod, iv, emb, dout, semi, seme):
    wid = lax.axis_index("s") * _NC + lax.axis_index("c")
    base = wid * _BW
    pltpu.sync_copy(tw_hbm.at[pl.ds(base, _BW)], tw_v)
    pltpu.async_copy(plen_hbm.at[tw_v], plen_v, seme[0]).wait()

    lane = lax.iota(jnp.int32, 16)

    lv0 = lane * _V
    lv1 = (lane + 8) * _V

    def idx_body(i, carry2):
        twi = plsc.load_gather(tw_v, [jnp.full((16,), i, jnp.int32)])
        fb = i * _L
        idx_v[pl.ds(fb, 16)] = lv0 + twi
        idx_v[pl.ds(fb + 8, 16)] = lv1 + twi
        return carry2

    lax.fori_loop(0, _BW, idx_body, 0)


    def p_stage(ch, k4, k2):
        coff = ch * _C
        idx = idx_v.at[pl.ds(coff * _L, _CL)]
        pltpu.async_copy(codes_hbm.at[idx], cod[k4], semi[k2])
        pltpu.async_copy(pts_hbm.at[idx], pts[k4], semi[k2])
        pltpu.async_copy(iv_hbm.at[pl.ds(base + coff, _C)], iv[k4], semi[k2])

    def r_stage(ch, k4, k2):
        coff = ch * _C
        idx = idx_v.at[pl.ds(coff * _L, _CL)]
        pltpu.make_async_copy(codes_hbm.at[idx], cod[k4], semi[k2]).wait()
        pltpu.make_async_copy(pts_hbm.at[idx], pts[k4], semi[k2]).wait()
        pltpu.make_async_copy(
            iv_hbm.at[pl.ds(base + coff, _C)], iv[k4], semi[k2]).wait()
        pltpu.async_copy(emb_hbm.at[pts[k4]], emb[k2], seme[k2])

    def c_stage(ch, k4, k2):
        coff = ch * _C
        goff = base + coff
        pltpu.make_async_copy(emb_hbm.at[pts[k4]], emb[k2], seme[k2]).wait()
        emb_v, iv_v, codes_v, dout_v = emb[k2], iv[k4], cod[k4], dout[k2]

        splat15 = jnp.full((16,), 15, jnp.int32)

        def samp_body(i, carry2):
            plen = plsc.load_gather(
                plen_v, [jnp.full((16,), coff + i, jnp.int32)])
            iv0 = iv_v[i, pl.ds(0, 16)]
            iv1 = iv_v[i, pl.ds(16, 16)]
            iv2 = iv_v[i, pl.ds(32, 16)]
            iv3 = iv_v[i, pl.ds(48, 16)]
            dbase = i * _L
            res0 = jnp.full((16,), _MASK_DOT)
            res1 = jnp.full((16,), _MASK_DOT)
            for l in range(_L):
                e0 = emb_v[dbase + l, pl.ds(0, 16)]
                e1 = emb_v[dbase + l, pl.ds(16, 16)]
                e2 = emb_v[dbase + l, pl.ds(32, 16)]
                e3 = emb_v[dbase + l, pl.ds(48, 16)]
                acc = (iv0 * e0 + iv1 * e1) + (iv2 * e2 + iv3 * e3)
                csum = plsc.cumsum(acc)
                tot = lax.gather(
                    csum, splat15[:, None],
                    lax.GatherDimensionNumbers(
                        offset_dims=(), collapsed_slice_dims=(0,),
                        start_index_map=(0,)),
                    (1,), mode=lax.GatherScatterMode.PROMISE_IN_BOUNDS)
                if l < 16:
                    res0 = jnp.where(lane == l, tot, res0)
                else:
                    res1 = jnp.where(lane == l - 16, tot, res1)
            m0 = lane < plen
            mh = (lane + 16) < plen
            dout_v[pl.ds(dbase, 16)] = jnp.where(m0, res0, _MASK_DOT)
            dout_v[pl.ds(dbase + 16, 16)] = jnp.where(mh, res1, _MASK_DOT)
            m1 = (lane + 8) < plen
            c0 = codes_v[pl.ds(dbase, 16)]
            c1 = codes_v[pl.ds(dbase + 8, 16)]
            codes_v[pl.ds(dbase, 16)] = jnp.where(m0, c0, _MASK_CODE)
            codes_v[pl.ds(dbase + 8, 16)] = jnp.where(m1, c1, _MASK_CODE)
            return carry2

        lax.fori_loop(0, _C, samp_body, 0)
        pltpu.sync_copy(dout_v.at[pl.ds(0, _CL)],
                        dots_out.at[pl.ds(goff * _L, _CL)])
        pltpu.sync_copy(codes_v, codes_out.at[pl.ds(goff * _L, _CL)])

    p_stage(0, 0, 0)
    p_stage(1, 1, 1)
    r_stage(0, 0, 0)

    def body(kk, carry):
        ch0 = kk * 4
        for j in range(4):
            ch = ch0 + j
            r_stage(ch + 1, (j + 1) % 4, (j + 1) % 2)
            p_stage(ch + 2, (j + 2) % 4, j % 2)
            c_stage(ch, j, j % 2)
        return carry

    lax.fori_loop(0, (_NCHUNK - 4) // 4, body, 0)
    for ch in range(_NCHUNK - 4, _NCHUNK):
        if ch + 1 < _NCHUNK:
            r_stage(ch + 1, (ch + 1) % 4, (ch + 1) % 2)
        if ch + 2 < _NCHUNK:
            p_stage(ch + 2, (ch + 2) % 4, ch % 2)
        c_stage(ch, ch % 4, ch % 2)


_sc_dots = functools.partial(
    pl.kernel,
    mesh=plsc.VectorSubcoreMesh(core_axis_name="c", subcore_axis_name="s"),
    out_type=(jax.ShapeDtypeStruct((_B * _L,), jnp.float32),
              jax.ShapeDtypeStruct((_B * _L,), jnp.float32)),
    scratch_types=[
        pltpu.VMEM((_BW,), jnp.int32),
        pltpu.VMEM((_BW,), jnp.int32),
        pltpu.VMEM((_BW * _L,), jnp.int32),
        [pltpu.VMEM((_CL,), jnp.int32)] * 4,
        [pltpu.VMEM((_CL,), jnp.float32)] * 4,
        [pltpu.VMEM((_C, _D), jnp.float32)] * 4,
        [pltpu.VMEM((_CL, _D), jnp.float32)] * 2,
        [pltpu.VMEM((_CL + 16,), jnp.float32)] * 2,
        [pltpu.SemaphoreType.DMA] * 2,
        [pltpu.SemaphoreType.DMA] * 2,
    ],
    compiler_params=pltpu.CompilerParams(needs_layout_passes=False,
                                         use_tc_tiling_on_sc=False),
)(_sc_body)


def _loss_body(d_ref, c_ref, o_ref):
    d = d_ref[...]
    c = c_ref[...]
    s = 1.0 / (1.0 + jnp.exp(-d))
    loss = c * jnp.log(s + 1e-10) + (1.0 - c) * jnp.log(1.0 - s + 1e-10)
    o_ref[0, 0] = -jnp.sum(loss)


def kernel(input_vectors, target_words, internal_embeddings, codes_table,
           points_table, path_lengths_table):
    tw = target_words.astype(jnp.int32)
    plens = path_lengths_table.astype(jnp.int32)
    codes_flat = codes_table.T.reshape(-1)
    pts_flat = points_table.T.astype(jnp.int32).reshape(-1)
    dots, codes = _sc_dots(input_vectors, tw, internal_embeddings,
                           codes_flat, pts_flat, plens)
    r = (_B * _L) // 128
    total = pl.pallas_call(
        _loss_body,
        out_shape=jax.ShapeDtypeStruct((1, 1), jnp.float32),
        out_specs=pl.BlockSpec(memory_space=pltpu.SMEM),
    )(dots.reshape(r, 128), codes.reshape(r, 128))
    return total[0, 0] / _B


# speedup vs baseline: 8.4727x; 1.0277x over previous
import functools

import jax
import jax.numpy as jnp
from jax import lax
from jax.experimental import pallas as pl
from jax.experimental.pallas import tpu as pltpu
from jax.experimental.pallas import tpu_sc as plsc

_V = 100000
_D = 64
_B = 16384
_L = 24
_NI = _V - 1

_NC = 2
_NS = 16
_NW = _NC * _NS
_BW = _B // _NW
_C = 16
_NCHUNK = _BW // _C
_CL = _C * _L

_MASK_DOT = 40.0
_MASK_CODE = 1.0


def _sc_body(iv_hbm, tw_hbm, emb_hbm, codes_hbm, pts_hbm, plen_hbm,
             dots_out, codes_out,
             tw_v, plen_v, idx_v, ivw, pts, cod, emb, dout, semi, seme, semv):
    wid = lax.axis_index("s") * _NC + lax.axis_index("c")
    base = wid * _BW
    pltpu.sync_copy(tw_hbm.at[pl.ds(base, _BW)], tw_v)
    for g in range(4):
        for d in range(g * 16, g * 16 + 16):
            pltpu.async_copy(iv_hbm.at[pl.ds(d * _B + base, _BW)],
                             ivw.at[pl.ds(d * _BW, _BW)], semv)
        for d in range(g * 16, g * 16 + 16):
            pltpu.make_async_copy(iv_hbm.at[pl.ds(d * _B + base, _BW)],
                                  ivw.at[pl.ds(d * _BW, _BW)], semv).wait()
    pltpu.async_copy(plen_hbm.at[tw_v], plen_v, seme[0]).wait()

    lane = lax.iota(jnp.int32, 16)

    lv0 = lane * _V
    lv1 = (lane + 8) * _V

    def idx_body(i, carry2):
        twi = plsc.load_gather(tw_v, [jnp.full((16,), i, jnp.int32)])
        fb = i * _L
        idx_v[pl.ds(fb, 16)] = lv0 + twi
        idx_v[pl.ds(fb + 8, 16)] = lv1 + twi
        return carry2

    lax.fori_loop(0, _BW, idx_body, 0)


    def p_stage(ch, k4, k2):
        coff = ch * _C
        idx = idx_v.at[pl.ds(coff * _L, _CL)]
        pltpu.async_copy(codes_hbm.at[idx], cod[k4], semi[k2])
        pltpu.async_copy(pts_hbm.at[idx], pts[k4], semi[k2])

    def r_stage(ch, k4, k2):
        coff = ch * _C
        idx = idx_v.at[pl.ds(coff * _L, _CL)]
        pltpu.make_async_copy(codes_hbm.at[idx], cod[k4], semi[k2]).wait()
        pltpu.make_async_copy(pts_hbm.at[idx], pts[k4], semi[k2]).wait()
        pltpu.async_copy(emb_hbm.at[pts[k4]], emb[k2], seme[k2])

    def c_stage(ch, k4, k2):
        coff = ch * _C
        goff = base + coff
        pltpu.make_async_copy(emb_hbm.at[pts[k4]], emb[k2], seme[k2]).wait()
        emb_v, codes_v, dout_v = emb[k2], cod[k4], dout[k2]

        splat15 = jnp.full((16,), 15, jnp.int32)

        lanebw = lane * _BW

        def samp_body(i, carry2):
            spl = jnp.full((16,), coff + i, jnp.int32)
            plen = plsc.load_gather(plen_v, [spl])
            ivx = lanebw + spl
            iv0 = plsc.load_gather(ivw, [ivx])
            iv1 = plsc.load_gather(ivw, [ivx + 16 * _BW])
            iv2 = plsc.load_gather(ivw, [ivx + 32 * _BW])
            iv3 = plsc.load_gather(ivw, [ivx + 48 * _BW])
            dbase = i * _L
            res0 = jnp.full((16,), _MASK_DOT)
            res1 = jnp.full((16,), _MASK_DOT)
            for l in range(_L):
                e0 = emb_v[dbase + l, pl.ds(0, 16)]
                e1 = emb_v[dbase + l, pl.ds(16, 16)]
                e2 = emb_v[dbase + l, pl.ds(32, 16)]
                e3 = emb_v[dbase + l, pl.ds(48, 16)]
                acc = (iv0 * e0 + iv1 * e1) + (iv2 * e2 + iv3 * e3)
                csum = plsc.cumsum(acc)
                tot = lax.gather(
                    csum, splat15[:, None],
                    lax.GatherDimensionNumbers(
                        offset_dims=(), collapsed_slice_dims=(0,),
                        start_index_map=(0,)),
                    (1,), mode=lax.GatherScatterMode.PROMISE_IN_BOUNDS)
                if l < 16:
                    res0 = jnp.where(lane == l, tot, res0)
                else:
                    res1 = jnp.where(lane == l - 16, tot, res1)
            m0 = lane < plen
            mh = (lane + 16) < plen
            dout_v[pl.ds(dbase, 16)] = jnp.where(m0, res0, _MASK_DOT)
            dout_v[pl.ds(dbase + 16, 16)] = jnp.where(mh, res1, _MASK_DOT)
            m1 = (lane + 8) < plen
            c0 = codes_v[pl.ds(dbase, 16)]
            c1 = codes_v[pl.ds(dbase + 8, 16)]
            codes_v[pl.ds(dbase, 16)] = jnp.where(m0, c0, _MASK_CODE)
            codes_v[pl.ds(dbase + 8, 16)] = jnp.where(m1, c1, _MASK_CODE)
            return carry2

        lax.fori_loop(0, _C, samp_body, 0)
        pltpu.sync_copy(dout_v.at[pl.ds(0, _CL)],
                        dots_out.at[pl.ds(goff * _L, _CL)])
        pltpu.sync_copy(codes_v, codes_out.at[pl.ds(goff * _L, _CL)])

    p_stage(0, 0, 0)
    p_stage(1, 1, 1)
    r_stage(0, 0, 0)

    def body(kk, carry):
        ch0 = kk * 4
        for j in range(4):
            ch = ch0 + j
            r_stage(ch + 1, (j + 1) % 4, (j + 1) % 2)
            p_stage(ch + 2, (j + 2) % 4, j % 2)
            c_stage(ch, j, j % 2)
        return carry

    lax.fori_loop(0, (_NCHUNK - 4) // 4, body, 0)
    for ch in range(_NCHUNK - 4, _NCHUNK):
        if ch + 1 < _NCHUNK:
            r_stage(ch + 1, (ch + 1) % 4, (ch + 1) % 2)
        if ch + 2 < _NCHUNK:
            p_stage(ch + 2, (ch + 2) % 4, ch % 2)
        c_stage(ch, ch % 4, ch % 2)


_sc_dots = functools.partial(
    pl.kernel,
    mesh=plsc.VectorSubcoreMesh(core_axis_name="c", subcore_axis_name="s"),
    out_type=(jax.ShapeDtypeStruct((_B * _L,), jnp.float32),
              jax.ShapeDtypeStruct((_B * _L,), jnp.float32)),
    scratch_types=[
        pltpu.VMEM((_BW,), jnp.int32),
        pltpu.VMEM((_BW,), jnp.int32),
        pltpu.VMEM((_BW * _L,), jnp.int32),
        pltpu.VMEM((_D * _BW,), jnp.float32),
        [pltpu.VMEM((_CL,), jnp.int32)] * 4,
        [pltpu.VMEM((_CL,), jnp.float32)] * 4,
        [pltpu.VMEM((_CL, _D), jnp.float32)] * 2,
        [pltpu.VMEM((_CL + 16,), jnp.float32)] * 2,
        [pltpu.SemaphoreType.DMA] * 2,
        [pltpu.SemaphoreType.DMA] * 2,
        pltpu.SemaphoreType.DMA,
    ],
    compiler_params=pltpu.CompilerParams(needs_layout_passes=False,
                                         use_tc_tiling_on_sc=False),
)(_sc_body)


def _loss_body(d_ref, c_ref, o_ref):
    d = d_ref[...]
    c = c_ref[...]
    s = 1.0 / (1.0 + jnp.exp(-d))
    loss = c * jnp.log(s + 1e-10) + (1.0 - c) * jnp.log(1.0 - s + 1e-10)
    o_ref[0, 0] = -jnp.sum(loss)


def kernel(input_vectors, target_words, internal_embeddings, codes_table,
           points_table, path_lengths_table):
    tw = target_words.astype(jnp.int32)
    plens = path_lengths_table.astype(jnp.int32)
    codes_flat = codes_table.T.reshape(-1)
    pts_flat = points_table.T.astype(jnp.int32).reshape(-1)
    iv_flat = input_vectors.T.reshape(-1)
    dots, codes = _sc_dots(iv_flat, tw, internal_embeddings,
                           codes_flat, pts_flat, plens)
    r = (_B * _L) // 128
    total = pl.pallas_call(
        _loss_body,
        out_shape=jax.ShapeDtypeStruct((1, 1), jnp.float32),
        out_specs=pl.BlockSpec(memory_space=pltpu.SMEM),
    )(dots.reshape(r, 128), codes.reshape(r, 128))
    return total[0, 0] / _B
